# baseline (speedup 1.0000x reference)
import functools

import jax
import jax.numpy as jnp
from jax import lax
from jax.experimental import pallas as pl
from jax.experimental.pallas import tpu as pltpu
from jax.experimental.pallas import tpu_sc as plsc

N = 320000
P = 10000
P_PAD = 10240
D = 128
RECON_W = 1.0
CONSIST_W = 0.5

NC = 2
NS = 16
NQ = 4
NPASS = NQ // NC
QROWS = P_PAD // NQ
TRASH = QROWS
ACC_ROWS = QROWS + 8
CHUNK = 128
UNIT = 2 * CHUNK
NGRP = CHUNK // 16
BATCH = 160
UPB = BATCH // 2
NCHUNKS_PAD = 2560
NBATCH = NCHUNKS_PAD // (NS * BATCH)
IDX_PAD = NCHUNKS_PAD * CHUNK
IDX_FILL = P_PAD
ROWS_PER_TILE = QROWS // NS
LCNT = QROWS + 16


def _sc_segment_sum(child_state, idx_padded):
    mesh = plsc.VectorSubcoreMesh(core_axis_name="c", subcore_axis_name="s",
                                  num_cores=NC)

    @functools.partial(
        pl.kernel,
        out_type=(
            jax.ShapeDtypeStruct((NQ, QROWS, D), jnp.float32),
            jax.ShapeDtypeStruct((NQ * NS * LCNT,), jnp.float32),
        ),
        mesh=mesh,
        scratch_types=(
            pltpu.VMEM_SHARED((ACC_ROWS, D), jnp.float32),
            pltpu.VMEM((BATCH * CHUNK,), jnp.int32),
            pltpu.VMEM((CHUNK,), jnp.int32),
            pltpu.VMEM((CHUNK,), jnp.int32),
            pltpu.VMEM((CHUNK,), jnp.int32),
            pltpu.VMEM((CHUNK,), jnp.int32),
            pltpu.VMEM((2, UNIT, D), jnp.float32),
            pltpu.VMEM((LCNT,), jnp.float32),
            pltpu.SMEM((UPB,), jnp.int32),
            pltpu.SemaphoreType.DMA,
            pltpu.SemaphoreType.DMA,
            pltpu.SemaphoreType.DMA,
            pltpu.SemaphoreType.DMA,
        ),
    )
    def k(cs_hbm, idx_hbm, sums_out, cnts_out, acc, idx_flat,
          ia0, ia1, ib0, ib1, rows, lcnt, ulist, rs0, rs1, ss0, ss1):
        c = lax.axis_index("c")
        s = lax.axis_index("s")
        base_row = s * ROWS_PER_TILE

        zv = jnp.zeros((16,), jnp.float32)
        iota = lax.iota(jnp.int32, 16)

        def rows_dma(g2, b):
            src = cs_hbm.at[pl.ds(g2 * CHUNK, UNIT)]

            @pl.when(b == 0)
            def _():
                pltpu.async_copy(src, rows.at[0], rs0)

            @pl.when(b == 1)
            def _():
                pltpu.async_copy(src, rows.at[1], rs1)

        def rows_wait(b):
            @pl.when(b == 0)
            def _():
                pltpu.make_async_copy(
                    cs_hbm.at[pl.ds(0, UNIT)], rows.at[0], rs0).wait()

            @pl.when(b == 1)
            def _():
                pltpu.make_async_copy(
                    cs_hbm.at[pl.ds(0, UNIT)], rows.at[1], rs1).wait()

        def scatter_wait(b):
            @pl.when(b == 0)
            def _():
                pltpu.make_async_copy(
                    rows.at[0, pl.ds(0, CHUNK)], acc.at[ia0], ss0).wait()
                pltpu.make_async_copy(
                    rows.at[0, pl.ds(0, CHUNK)], acc.at[ia1], ss0).wait()

            @pl.when(b == 1)
            def _():
                pltpu.make_async_copy(
                    rows.at[1, pl.ds(0, CHUNK)], acc.at[ib0], ss1).wait()
                pltpu.make_async_copy(
                    rows.at[1, pl.ds(0, CHUNK)], acc.at[ib1], ss1).wait()

        for q in range(NPASS):
            qlo = (c * NPASS + q) * QROWS
            qhi = qlo + QROWS

            def fill_z(i, _):
                rows[0, i // 8, pl.ds((i % 8) * 16, 16)] = zv
                return 0
            lax.fori_loop(0, ROWS_PER_TILE * 8, fill_z, 0)

            def fill_lc(i, _):
                lcnt[pl.ds(i * 16, 16)] = zv
                return 0
            lax.fori_loop(0, LCNT // 16, fill_lc, 0)
            pltpu.sync_copy(rows.at[0, pl.ds(0, ROWS_PER_TILE)],
                            acc.at[pl.ds(base_row, ROWS_PER_TILE)])
            plsc.subcore_barrier()

            def count_group(voff, _):
                vraw = idx_flat[pl.ds(voff, 16)] - qlo
                ok = jnp.logical_and(vraw[0] >= 0, vraw[15] < QROWS)
                v2 = jnp.where(
                    jnp.logical_and(vraw >= 0, vraw < QROWS),
                    vraw, jnp.full((16,), TRASH, jnp.int32))

                def per_lane(_):
                    for j in range(16):
                        ej = v2[j]
                        bj = lax.shift_right_arithmetic(ej, 4)
                        w = lcnt[pl.ds(bj * 16, 16)]
                        lcnt[pl.ds(bj * 16, 16)] = w + jnp.where(
                            iota == ej - bj * 16, 1.0, 0.0)
                    return 0

                def in_range(_):
                    e0 = vraw[0]
                    e15 = vraw[15]
                    bA = lax.shift_right_arithmetic(e0, 4)

                    def const_group(_):
                        w = lcnt[pl.ds(bA * 16, 16)]
                        lcnt[pl.ds(bA * 16, 16)] = w + jnp.where(
                            iota == e0 - bA * 16, 16.0, 0.0)
                        return 0

                    def two_block(_):
                        a = jnp.zeros((16,), jnp.float32)
                        bb = jnp.zeros((16,), jnp.float32)
                        for j in range(16):
                            ej = vraw[j]
                            a = a + jnp.where(
                                iota == ej - bA * 16, 1.0, 0.0)
                            bb = bb + jnp.where(
                                iota == ej - (bA + 1) * 16, 1.0, 0.0)
                        wa = lcnt[pl.ds(bA * 16, 16)]
                        lcnt[pl.ds(bA * 16, 16)] = wa + a
                        wb = lcnt[pl.ds((bA + 1) * 16, 16)]
                        lcnt[pl.ds((bA + 1) * 16, 16)] = wb + bb
                        return 0
                    return lax.cond(e0 == e15, const_group,
                                    lambda _: lax.cond(
                                        lax.shift_right_arithmetic(e15, 4)
                                        <= bA + 1,
                                        two_block, per_lane, 0), 0)

                lax.cond(ok, in_range, per_lane, 0)
                return 0

            def remap_half(uoff, idx2_b):
                for grp in range(NGRP):
                    v = idx_flat[pl.ds(uoff + grp * 16, 16)] - qlo
                    okv = jnp.logical_and(v >= 0, v < QROWS)
                    idx2_b[pl.ds(grp * 16, 16)] = jnp.where(
                        okv, v, jnp.full((16,), TRASH, jnp.int32))

            def batch_body(j, _):
                g0 = (j * NS + s) * BATCH
                pltpu.sync_copy(idx_hbm.at[pl.ds(g0 * CHUNK, BATCH * CHUNK)],
                                idx_flat)
                bmn = idx_flat[pl.ds(0, 16)][0]
                bmx = idx_flat[pl.ds(BATCH * CHUNK - 16, 16)][15]

                def batch_active(_):
                    def scan_unit(u, na):
                        umn = idx_flat[pl.ds(u * UNIT, 16)][0]
                        umx = idx_flat[pl.ds((u + 1) * UNIT - 16, 16)][15]
                        act = jnp.logical_and(umx >= qlo, umn < qhi)
                        ulist[na] = u
                        return na + act.astype(jnp.int32)
                    na = lax.fori_loop(0, UPB, scan_unit, jnp.int32(0))

                    @pl.when(na >= 1)
                    def _():
                        rows_dma(g0 + ulist[0] * 2, 0)

                    def unit_body(i, _):
                        b = lax.rem(i, 2)
                        nb = 1 - b
                        u = ulist[i]
                        uoff = u * UNIT

                        @pl.when(i + 1 < na)
                        def _():
                            @pl.when(i >= 1)
                            def _():
                                scatter_wait(nb)
                            rows_dma(g0 + ulist[i + 1] * 2, nb)

                        rows_wait(b)

                        @pl.when(b == 0)
                        def _():
                            remap_half(uoff, ia0)
                            remap_half(uoff + CHUNK, ia1)
                            pltpu.async_copy(rows.at[0, pl.ds(0, CHUNK)],
                                             acc.at[ia0], ss0, add=True)
                            pltpu.async_copy(rows.at[0, pl.ds(CHUNK, CHUNK)],
                                             acc.at[ia1], ss0, add=True)

                        @pl.when(b == 1)
                        def _():
                            remap_half(uoff, ib0)
                            remap_half(uoff + CHUNK, ib1)
                            pltpu.async_copy(rows.at[1, pl.ds(0, CHUNK)],
                                             acc.at[ib0], ss1, add=True)
                            pltpu.async_copy(rows.at[1, pl.ds(CHUNK, CHUNK)],
                                             acc.at[ib1], ss1, add=True)

                        lax.fori_loop(0, 2 * NGRP,
                                      lambda gg, uu: count_group(
                                          uoff + gg * 16, uu),
                                      0)
                        return 0
                    lax.fori_loop(0, na, unit_body, 0)

                    @pl.when(na >= 1)
                    def _():
                        scatter_wait(lax.rem(na - 1, 2))

                    @pl.when(na >= 2)
                    def _():
                        scatter_wait(lax.rem(na, 2))
                    return 0
                lax.cond(jnp.logical_and(bmx >= qlo, bmn < qhi),
                         batch_active, lambda _: 0, 0)
                return 0
            lax.fori_loop(0, NBATCH, batch_body, 0)
            plsc.subcore_barrier()

            qi = c * NPASS + q
            pltpu.sync_copy(acc.at[pl.ds(base_row, ROWS_PER_TILE)],
                            sums_out.at[qi, pl.ds(base_row, ROWS_PER_TILE)])
            pltpu.sync_copy(lcnt,
                            cnts_out.at[pl.ds((qi * NS + s) * LCNT, LCNT)])

    return k(child_state, idx_padded)


RB = 16000


def _recon_sse(reconstructed, target):
    def body(r_ref, t_ref, o_ref):
        @pl.when(pl.program_id(0) == 0)
        def _():
            o_ref[0, 0] = 0.0
        d = r_ref[...] - t_ref[...]
        o_ref[0, 0] += jnp.sum(d * d)

    return pl.pallas_call(
        body,
        grid=(N // RB,),
        in_specs=[
            pl.BlockSpec((RB, D), lambda i: (i, 0)),
            pl.BlockSpec((RB, D), lambda i: (i, 0)),
        ],
        out_specs=pl.BlockSpec(memory_space=pltpu.SMEM),
        out_shape=jax.ShapeDtypeStruct((1, 1), jnp.float32),
    )(reconstructed, target)


def _consist_sse(sums4, cnts4, parent4):
    def body(s_ref, c_ref, p_ref, o_ref):
        @pl.when(pl.program_id(0) == 0)
        def _():
            o_ref[0, 0] = 0.0
        cnt = jnp.sum(c_ref[0], axis=1, keepdims=True)
        agg = s_ref[0] / jnp.maximum(cnt, 1.0)
        d = agg - p_ref[0]
        o_ref[0, 0] += jnp.sum(d * d)

    return pl.pallas_call(
        body,
        grid=(NQ,),
        in_specs=[
            pl.BlockSpec((1, QROWS, D), lambda i: (i, 0, 0)),
            pl.BlockSpec((1, QROWS, NS), lambda i: (i, 0, 0)),
            pl.BlockSpec((1, QROWS, D), lambda i: (i, 0, 0)),
        ],
        out_specs=pl.BlockSpec(memory_space=pltpu.SMEM),
        out_shape=jax.ShapeDtypeStruct((1, 1), jnp.float32),
    )(sums4, cnts4, parent4)


def kernel(reconstructed, target, child_state, parent_state,
           child_to_parent_idx, num_parents):
    idx = child_to_parent_idx.astype(jnp.int32)
    idx_padded = jnp.pad(idx, (0, IDX_PAD - N), constant_values=IDX_FILL)
    parent4 = jnp.pad(parent_state,
                      ((0, P_PAD - P), (0, 0))).reshape(NQ, QROWS, D)
    sums4, cnts_flat = _sc_segment_sum(child_state, idx_padded)
    cnts4 = cnts_flat.reshape(NQ, NS, LCNT)[:, :, :QROWS].transpose(0, 2, 1)
    recon_sse = _recon_sse(reconstructed, target)[0, 0]
    consist_sse = _consist_sse(sums4, cnts4, parent4)[0, 0]
    recon_loss = recon_sse / (N * D)
    consist_loss = consist_sse / (P * D)
    return RECON_W * recon_loss + CONSIST_W * consist_loss


# speedup vs baseline: 1.8696x; 1.8696x over previous
import functools

import jax
import jax.numpy as jnp
from jax import lax
from jax.experimental import pallas as pl
from jax.experimental.pallas import tpu as pltpu
from jax.experimental.pallas import tpu_sc as plsc

N = 320000
P = 10000
P_PAD = 10240
D = 128
RECON_W = 1.0
CONSIST_W = 0.5

NC = 2
NS = 16
NQ = 4
NPASS = NQ // NC
QROWS = P_PAD // NQ
TRASH = QROWS
ACC_ROWS = QROWS + 8
CHUNK = 128
UNIT = 2 * CHUNK
NGRP = CHUNK // 16
BATCH = 16
UPB = BATCH // 2
NCHUNKS_PAD = 2560
NBATCH = NCHUNKS_PAD // (NS * BATCH)
NUNITS = NCHUNKS_PAD // (2 * NS)
IDX_PAD = NCHUNKS_PAD * CHUNK
IDX_FILL = P_PAD
ROWS_PER_TILE = QROWS // NS
LCNT = QROWS + 16


def _sc_segment_sum(child_state, idx_padded):
    mesh = plsc.VectorSubcoreMesh(core_axis_name="c", subcore_axis_name="s",
                                  num_cores=NC)

    @functools.partial(
        pl.kernel,
        out_type=(
            jax.ShapeDtypeStruct((NQ, QROWS, D), jnp.float32),
            jax.ShapeDtypeStruct((NQ * NS * LCNT,), jnp.float32),
        ),
        mesh=mesh,
        scratch_types=(
            pltpu.VMEM_SHARED((ACC_ROWS, D), jnp.float32),
            pltpu.VMEM((NCHUNKS_PAD * CHUNK // NS,), jnp.int32),
            pltpu.VMEM((CHUNK,), jnp.int32),
            pltpu.VMEM((CHUNK,), jnp.int32),
            pltpu.VMEM((CHUNK,), jnp.int32),
            pltpu.VMEM((CHUNK,), jnp.int32),
            pltpu.VMEM((2, UNIT, D), jnp.float32),
            pltpu.VMEM((LCNT,), jnp.float32),
            pltpu.SMEM((NUNITS,), jnp.int32),
            pltpu.SemaphoreType.DMA,
            pltpu.SemaphoreType.DMA,
            pltpu.SemaphoreType.DMA,
            pltpu.SemaphoreType.DMA,
            pltpu.SemaphoreType.DMA,
        ),
    )
    def k(cs_hbm, idx_hbm, sums_out, cnts_out, acc, idx_all,
          ia0, ia1, ib0, ib1, rows, lcnt, ulist, isem, rs0, rs1, ss0, ss1):
        c = lax.axis_index("c")
        s = lax.axis_index("s")
        base_row = s * ROWS_PER_TILE

        zv = jnp.zeros((16,), jnp.float32)
        iota = lax.iota(jnp.int32, 16)

        def rows_dma(g2, b):
            src = cs_hbm.at[pl.ds(g2 * CHUNK, UNIT)]

            @pl.when(b == 0)
            def _():
                pltpu.async_copy(src, rows.at[0], rs0)

            @pl.when(b == 1)
            def _():
                pltpu.async_copy(src, rows.at[1], rs1)

        def rows_wait(b):
            @pl.when(b == 0)
            def _():
                pltpu.make_async_copy(
                    cs_hbm.at[pl.ds(0, UNIT)], rows.at[0], rs0).wait()

            @pl.when(b == 1)
            def _():
                pltpu.make_async_copy(
                    cs_hbm.at[pl.ds(0, UNIT)], rows.at[1], rs1).wait()

        def scatter_wait(b):
            @pl.when(b == 0)
            def _():
                pltpu.make_async_copy(
                    rows.at[0, pl.ds(0, CHUNK)], acc.at[ia0], ss0).wait()
                pltpu.make_async_copy(
                    rows.at[0, pl.ds(0, CHUNK)], acc.at[ia1], ss0).wait()

            @pl.when(b == 1)
            def _():
                pltpu.make_async_copy(
                    rows.at[1, pl.ds(0, CHUNK)], acc.at[ib0], ss1).wait()
                pltpu.make_async_copy(
                    rows.at[1, pl.ds(0, CHUNK)], acc.at[ib1], ss1).wait()

        for q in range(NPASS):
            qlo = (c * NPASS + q) * QROWS
            qhi = qlo + QROWS

            def fill_z(i, _):
                rows[0, i // 8, pl.ds((i % 8) * 16, 16)] = zv
                return 0
            lax.fori_loop(0, ROWS_PER_TILE * 8, fill_z, 0)

            def fill_lc(i, _):
                lcnt[pl.ds(i * 16, 16)] = zv
                return 0
            lax.fori_loop(0, LCNT // 16, fill_lc, 0)
            pltpu.sync_copy(rows.at[0, pl.ds(0, ROWS_PER_TILE)],
                            acc.at[pl.ds(base_row, ROWS_PER_TILE)])
            plsc.subcore_barrier()

            def count_group(voff, _):
                vraw = idx_all[pl.ds(voff, 16)] - qlo
                ok = jnp.logical_and(vraw[0] >= 0, vraw[15] < QROWS)
                v2 = jnp.where(
                    jnp.logical_and(vraw >= 0, vraw < QROWS),
                    vraw, jnp.full((16,), TRASH, jnp.int32))

                def per_lane(_):
                    for j in range(16):
                        ej = v2[j]
                        bj = lax.shift_right_arithmetic(ej, 4)
                        w = lcnt[pl.ds(bj * 16, 16)]
                        lcnt[pl.ds(bj * 16, 16)] = w + jnp.where(
                            iota == ej - bj * 16, 1.0, 0.0)
                    return 0

                def in_range(_):
                    e0 = vraw[0]
                    e15 = vraw[15]
                    bA = lax.shift_right_arithmetic(e0, 4)

                    def const_group(_):
                        w = lcnt[pl.ds(bA * 16, 16)]
                        lcnt[pl.ds(bA * 16, 16)] = w + jnp.where(
                            iota == e0 - bA * 16, 16.0, 0.0)
                        return 0

                    def two_block(_):
                        a = jnp.zeros((16,), jnp.float32)
                        bb = jnp.zeros((16,), jnp.float32)
                        for j in range(16):
                            ej = vraw[j]
                            a = a + jnp.where(
                                iota == ej - bA * 16, 1.0, 0.0)
                            bb = bb + jnp.where(
                                iota == ej - (bA + 1) * 16, 1.0, 0.0)
                        wa = lcnt[pl.ds(bA * 16, 16)]
                        lcnt[pl.ds(bA * 16, 16)] = wa + a
                        wb = lcnt[pl.ds((bA + 1) * 16, 16)]
                        lcnt[pl.ds((bA + 1) * 16, 16)] = wb + bb
                        return 0
                    return lax.cond(e0 == e15, const_group,
                                    lambda _: lax.cond(
                                        lax.shift_right_arithmetic(e15, 4)
                                        <= bA + 1,
                                        two_block, per_lane, 0), 0)

                lax.cond(ok, in_range, per_lane, 0)
                return 0

            def remap_half(uoff, idx2_b):
                for grp in range(NGRP):
                    v = idx_all[pl.ds(uoff + grp * 16, 16)] - qlo
                    okv = jnp.logical_and(v >= 0, v < QROWS)
                    idx2_b[pl.ds(grp * 16, 16)] = jnp.where(
                        okv, v, jnp.full((16,), TRASH, jnp.int32))

            for j in range(NBATCH):
                pltpu.async_copy(
                    idx_hbm.at[pl.ds((j * NS + s) * BATCH * CHUNK,
                                     BATCH * CHUNK)],
                    idx_all.at[pl.ds(j * BATCH * CHUNK, BATCH * CHUNK)],
                    isem)
            for j in range(NBATCH):
                pltpu.make_async_copy(
                    idx_hbm.at[pl.ds(0, BATCH * CHUNK)],
                    idx_all.at[pl.ds(0, BATCH * CHUNK)], isem).wait()

            def scan_unit(u, na):
                umn = idx_all[pl.ds(u * UNIT, 16)][0]
                umx = idx_all[pl.ds((u + 1) * UNIT - 16, 16)][15]
                act = jnp.logical_and(umx >= qlo, umn < qhi)
                ulist[na] = u
                return na + act.astype(jnp.int32)
            na = lax.fori_loop(0, NUNITS, scan_unit, jnp.int32(0))

            def chunk_of_unit(u):
                return ((u // UPB) * NS + s) * BATCH + lax.rem(u, UPB) * 2

            @pl.when(na >= 1)
            def _():
                rows_dma(chunk_of_unit(ulist[0]), 0)

            def unit_body(i, _):
                b = lax.rem(i, 2)
                nb = 1 - b
                u = ulist[i]
                uoff = u * UNIT

                @pl.when(i + 1 < na)
                def _():
                    @pl.when(i >= 1)
                    def _():
                        scatter_wait(nb)
                    rows_dma(chunk_of_unit(ulist[i + 1]), nb)

                rows_wait(b)

                @pl.when(b == 0)
                def _():
                    remap_half(uoff, ia0)
                    remap_half(uoff + CHUNK, ia1)
                    pltpu.async_copy(rows.at[0, pl.ds(0, CHUNK)],
                                     acc.at[ia0], ss0, add=True)
                    pltpu.async_copy(rows.at[0, pl.ds(CHUNK, CHUNK)],
                                     acc.at[ia1], ss0, add=True)

                @pl.when(b == 1)
                def _():
                    remap_half(uoff, ib0)
                    remap_half(uoff + CHUNK, ib1)
                    pltpu.async_copy(rows.at[1, pl.ds(0, CHUNK)],
                                     acc.at[ib0], ss1, add=True)
                    pltpu.async_copy(rows.at[1, pl.ds(CHUNK, CHUNK)],
                                     acc.at[ib1], ss1, add=True)

                lax.fori_loop(0, 2 * NGRP,
                              lambda gg, uu: count_group(uoff + gg * 16, uu),
                              0)
                return 0
            lax.fori_loop(0, na, unit_body, 0)

            @pl.when(na >= 1)
            def _():
                scatter_wait(lax.rem(na - 1, 2))

            @pl.when(na >= 2)
            def _():
                scatter_wait(lax.rem(na, 2))
            plsc.subcore_barrier()

            qi = c * NPASS + q
            pltpu.sync_copy(acc.at[pl.ds(base_row, ROWS_PER_TILE)],
                            sums_out.at[qi, pl.ds(base_row, ROWS_PER_TILE)])
            pltpu.sync_copy(lcnt,
                            cnts_out.at[pl.ds((qi * NS + s) * LCNT, LCNT)])

    return k(child_state, idx_padded)


RB = 16000


def _recon_sse(reconstructed, target):
    def body(r_ref, t_ref, o_ref):
        @pl.when(pl.program_id(0) == 0)
        def _():
            o_ref[0, 0] = 0.0
        d = r_ref[...] - t_ref[...]
        o_ref[0, 0] += jnp.sum(d * d)

    return pl.pallas_call(
        body,
        grid=(N // RB,),
        in_specs=[
            pl.BlockSpec((RB, D), lambda i: (i, 0)),
            pl.BlockSpec((RB, D), lambda i: (i, 0)),
        ],
        out_specs=pl.BlockSpec(memory_space=pltpu.SMEM),
        out_shape=jax.ShapeDtypeStruct((1, 1), jnp.float32),
    )(reconstructed, target)


def _consist_sse(sums4, cnts4, parent4):
    def body(s_ref, c_ref, p_ref, o_ref):
        @pl.when(pl.program_id(0) == 0)
        def _():
            o_ref[0, 0] = 0.0
        cnt = jnp.sum(c_ref[0], axis=1, keepdims=True)
        agg = s_ref[0] / jnp.maximum(cnt, 1.0)
        d = agg - p_ref[0]
        o_ref[0, 0] += jnp.sum(d * d)

    return pl.pallas_call(
        body,
        grid=(NQ,),
        in_specs=[
            pl.BlockSpec((1, QROWS, D), lambda i: (i, 0, 0)),
            pl.BlockSpec((1, QROWS, NS), lambda i: (i, 0, 0)),
            pl.BlockSpec((1, QROWS, D), lambda i: (i, 0, 0)),
        ],
        out_specs=pl.BlockSpec(memory_space=pltpu.SMEM),
        out_shape=jax.ShapeDtypeStruct((1, 1), jnp.float32),
    )(sums4, cnts4, parent4)


def kernel(reconstructed, target, child_state, parent_state,
           child_to_parent_idx, num_parents):
    idx = child_to_parent_idx.astype(jnp.int32)
    idx_padded = jnp.pad(idx, (0, IDX_PAD - N), constant_values=IDX_FILL)
    parent4 = jnp.pad(parent_state,
                      ((0, P_PAD - P), (0, 0))).reshape(NQ, QROWS, D)
    sums4, cnts_flat = _sc_segment_sum(child_state, idx_padded)
    cnts4 = cnts_flat.reshape(NQ, NS, LCNT)[:, :, :QROWS].transpose(0, 2, 1)
    recon_sse = _recon_sse(reconstructed, target)[0, 0]
    consist_sse = _consist_sse(sums4, cnts4, parent4)[0, 0]
    recon_loss = recon_sse / (N * D)
    consist_loss = consist_sse / (P * D)
    return RECON_W * recon_loss + CONSIST_W * consist_loss
